# baseline (speedup 1.0000x reference)
import functools

import jax
import jax.numpy as jnp
from jax import lax
from jax.experimental import pallas as pl
from jax.experimental.pallas import tpu as pltpu
from jax.experimental.pallas import tpu_sc as plsc

N_NODES = 10000
N_EDGES = 320000
D = 128
L = 16
NC = 2
NS = 16
NW = NC * NS
E_PER_W = N_EDGES // NW
C = 64
NFULL = E_PER_W // C
TAIL = E_PER_W - NFULL * C
GROUPS = C // L
UNROLL = 8
MR = 3
MI = 4
MV = 3
UC = 12

_mesh = plsc.VectorSubcoreMesh(
    core_axis_name="c", subcore_axis_name="s", num_cores=NC, num_subcores=NS
)


@functools.partial(
    pl.kernel,
    out_type=jax.ShapeDtypeStruct((NC, N_NODES, D), jnp.float32),
    mesh=_mesh,
    scratch_types=[
        [pltpu.VMEM((C,), jnp.int32) for _ in range(MI)],
        [pltpu.VMEM((C,), jnp.int32) for _ in range(MI)],
        [pltpu.VMEM((C,), jnp.float32) for _ in range(MV)],
        [pltpu.VMEM((C, D), jnp.float32) for _ in range(MR)],
        [pltpu.VMEM((C, D), jnp.float32) for _ in range(MR)],
        pltpu.VMEM((TAIL,), jnp.int32),
        pltpu.VMEM((TAIL,), jnp.int32),
        pltpu.VMEM((TAIL,), jnp.float32),
        pltpu.VMEM_SHARED((N_NODES, D), jnp.float32),
        [pltpu.SemaphoreType.DMA for _ in range(MI)],
        [pltpu.SemaphoreType.DMA for _ in range(MR)],
        [pltpu.SemaphoreType.DMA for _ in range(MR)],
    ],
    compiler_params=pltpu.CompilerParams(needs_layout_passes=False),
)
def _sc_propagate(emb, src, dst, ev, out, sidx, didx, evv, srows, drows,
                  tsi, tdi, tev, acc, sem_m, sem_g, sem_s):
    cid = lax.axis_index("c")
    sid = lax.axis_index("s")
    wid = sid * NC + cid
    ebase = wid * E_PER_W
    lane = lax.iota(jnp.int32, L)
    zero16 = jnp.zeros((L,), jnp.float32)

    for i in range(C):
        for j in range(D // L):
            srows[0][i, pl.ds(j * L, L)] = zero16
    row0 = sid * 624
    for i in range(624 // C):
        pltpu.sync_copy(srows[0], acc.at[pl.ds(row0 + i * C, C)])
    pltpu.sync_copy(srows[0].at[pl.ds(0, 624 - (624 // C) * C)],
                    acc.at[pl.ds(row0 + (624 // C) * C, 624 - (624 // C) * C)])

    @pl.when(sid == NS - 1)
    def _zero_tail():
        pltpu.sync_copy(srows[0].at[pl.ds(0, 16)], acc.at[pl.ds(9984, 16)])

    plsc.subcore_barrier()

    def issue_ids(c, mi, mv):
        off = ebase + c * C
        pltpu.async_copy(src.at[pl.ds(off, C)], sidx[mi], sem_m[mi])
        pltpu.async_copy(dst.at[pl.ds(off, C)], didx[mi], sem_m[mi])
        pltpu.async_copy(ev.at[pl.ds(off, C)], evv[mv], sem_m[mi])

    def wait_ids(c, mi, mv):
        off = ebase + c * C
        pltpu.make_async_copy(src.at[pl.ds(off, C)], sidx[mi], sem_m[mi]).wait()
        pltpu.make_async_copy(dst.at[pl.ds(off, C)], didx[mi], sem_m[mi]).wait()
        pltpu.make_async_copy(ev.at[pl.ds(off, C)], evv[mv], sem_m[mi]).wait()

    def issue_gathers(mi, mr):
        for q in range(4):
            qs = pl.ds(q * L, L)
            pltpu.async_copy(emb.at[sidx[mi].at[qs]], srows[mr].at[qs],
                             sem_g[mr])
            pltpu.async_copy(emb.at[didx[mi].at[qs]], drows[mr].at[qs],
                             sem_g[mr])

    def wait_gathers(mi, mr):
        for q in range(4):
            qs = pl.ds(q * L, L)
            pltpu.make_async_copy(emb.at[sidx[mi].at[qs]], srows[mr].at[qs],
                                  sem_g[mr]).wait()
            pltpu.make_async_copy(emb.at[didx[mi].at[qs]], drows[mr].at[qs],
                                  sem_g[mr]).wait()

    def issue_scatter(mi, mr):
        pltpu.async_copy(srows[mr], acc.at[sidx[mi]], sem_s[mr], add=True)

    def wait_scatter(mi, mr):
        pltpu.make_async_copy(srows[mr], acc.at[sidx[mi]], sem_s[mr]).wait()

    def softplus_mat(dot, evvec):
        t = 4.0 - 4.0 * dot
        u = jnp.exp(-jnp.abs(t))
        w = u * (1.0 + u * (-0.5 + u * (1.0 / 3.0 + u * -0.25)))
        for _ in range(3):
            w = w - 1.0 + (1.0 + u) * jnp.exp(-w)
        sp = jnp.maximum(t, 0.0) + w
        return (0.5 * jnp.exp(t) * sp) * evvec

    def compute(mr, mv, groups):
        sr, dr, evr = srows[mr], drows[mr], evv[mv]

        def group_body(g, _):
            evec = lane + g * L

            def dot_body(kk, accs):
                a0, a1 = accs
                for j in range(UNROLL):
                    ks = jnp.full((L,), kk * UNROLL + j, jnp.int32)
                    sv = plsc.load_gather(sr, [evec, ks])
                    dv = plsc.load_gather(dr, [evec, ks])
                    if j % 2 == 0:
                        a0 = a0 + sv * dv
                    else:
                        a1 = a1 + sv * dv
                return a0, a1

            a0, a1 = lax.fori_loop(0, D // UNROLL, dot_body, (zero16, zero16))
            dot = (a0 + a1) * (1.0 / D)
            mat = softplus_mat(dot, evr[pl.ds(g * L, L)])

            def msg_body(kk, _):
                for j in range(UNROLL):
                    ks = jnp.full((L,), kk * UNROLL + j, jnp.int32)
                    dv = plsc.load_gather(dr, [evec, ks])
                    plsc.store_scatter(sr, [evec, ks], dv * mat)
                return 0

            lax.fori_loop(0, D // UNROLL, msg_body, 0)
            return 0

        lax.fori_loop(0, groups, group_body, 0)

    issue_ids(0, 0, 0)
    issue_ids(1, 1, 1)
    wait_ids(0, 0, 0)
    issue_gathers(0, 0)

    def body(i, carry):
        c0 = UC * i
        for j in range(UC):
            c = c0 + j
            mr, mi, mv = j % MR, j % MI, j % MV
            mr1, mi1, mv1 = (j + 1) % MR, (j + 1) % MI, (j + 1) % MV
            mi2, mv2 = (j + 2) % MI, (j + 2) % MV

            if j >= 2:
                wait_scatter((j - 2) % MI, (j - 2) % MR)
            else:
                @pl.when(i > 0)
                def _ws():
                    wait_scatter((j - 2) % MI, (j - 2) % MR)

            with jax.named_scope("stage"):
                @pl.when(c + 1 < NFULL)
                def _g1():
                    wait_ids(c + 1, mi1, mv1)
                    issue_gathers(mi1, mr1)

                @pl.when(c + 2 < NFULL)
                def _s2():
                    issue_ids(c + 2, mi2, mv2)

            with jax.named_scope("wgath"):
                wait_gathers(mi, mr)
            with jax.named_scope("comp"):
                compute(mr, mv, GROUPS)
            with jax.named_scope("scat"):
                issue_scatter(mi, mr)
        return carry

    lax.fori_loop(0, NFULL // UC, body, 0)
    wait_scatter((NFULL - 2) % MI, (NFULL - 2) % MR)
    wait_scatter((NFULL - 1) % MI, (NFULL - 1) % MR)

    toff = ebase + NFULL * C
    pltpu.sync_copy(src.at[pl.ds(toff, TAIL)], tsi)
    pltpu.sync_copy(dst.at[pl.ds(toff, TAIL)], tdi)
    pltpu.sync_copy(ev.at[pl.ds(toff, TAIL)], tev)
    cp_s = pltpu.async_copy(emb.at[tsi], srows[0].at[pl.ds(0, TAIL)], sem_g[0])
    cp_d = pltpu.async_copy(emb.at[tdi], drows[0].at[pl.ds(0, TAIL)], sem_g[0])
    cp_s.wait()
    cp_d.wait()
    evec = lane

    def tail_dot(kk, accs):
        a0, a1 = accs
        for j in range(UNROLL):
            ks = jnp.full((L,), kk * UNROLL + j, jnp.int32)
            sv = plsc.load_gather(srows[0], [evec, ks])
            dv = plsc.load_gather(drows[0], [evec, ks])
            if j % 2 == 0:
                a0 = a0 + sv * dv
            else:
                a1 = a1 + sv * dv
        return a0, a1

    a0, a1 = lax.fori_loop(0, D // UNROLL, tail_dot, (zero16, zero16))
    mat = softplus_mat((a0 + a1) * (1.0 / D), tev[...])

    def tail_msg(kk, _):
        for j in range(UNROLL):
            ks = jnp.full((L,), kk * UNROLL + j, jnp.int32)
            dv = plsc.load_gather(drows[0], [evec, ks])
            plsc.store_scatter(srows[0], [evec, ks], dv * mat)
        return 0

    lax.fori_loop(0, D // UNROLL, tail_msg, 0)
    pltpu.sync_copy(srows[0].at[pl.ds(0, TAIL)], acc.at[tsi], add=True)

    plsc.subcore_barrier()

    pltpu.sync_copy(acc.at[pl.ds(row0, 624)], out.at[cid, pl.ds(row0, 624)])

    @pl.when(sid == NS - 1)
    def _write_tail():
        pltpu.sync_copy(acc.at[pl.ds(9984, 16)], out.at[cid, pl.ds(9984, 16)])


def _combine_body(p_ref, o_ref):
    o_ref[...] = p_ref[0] + p_ref[1]


def _combine(parts):
    return pl.pallas_call(
        _combine_body,
        out_shape=jax.ShapeDtypeStruct((N_NODES, D), jnp.float32),
        grid=(10,),
        in_specs=[pl.BlockSpec((NC, N_NODES // 10, D), lambda i: (0, i, 0))],
        out_specs=pl.BlockSpec((N_NODES // 10, D), lambda i: (i, 0)),
    )(parts)


def kernel(emb, edge_index, edge_values):
    src = edge_index[0]
    dst = edge_index[1]
    parts = _sc_propagate(emb, src, dst, edge_values)
    return _combine(parts)


# speedup vs baseline: 1.0006x; 1.0006x over previous
import functools

import jax
import jax.numpy as jnp
from jax import lax
from jax.experimental import pallas as pl
from jax.experimental.pallas import tpu as pltpu
from jax.experimental.pallas import tpu_sc as plsc

N_NODES = 10000
N_EDGES = 320000
D = 128
L = 16
NC = 2
NS = 16
NW = NC * NS
E_PER_W = N_EDGES // NW
C = 64
NFULL = E_PER_W // C
TAIL = E_PER_W - NFULL * C
GROUPS = C // L
UNROLL = 8
MR = 3
MI = 4
MV = 3
UC = 12

_mesh = plsc.VectorSubcoreMesh(
    core_axis_name="c", subcore_axis_name="s", num_cores=NC, num_subcores=NS
)


@functools.partial(
    pl.kernel,
    out_type=jax.ShapeDtypeStruct((NC, N_NODES, D), jnp.float32),
    mesh=_mesh,
    scratch_types=[
        [pltpu.VMEM((C,), jnp.int32) for _ in range(MI)],
        [pltpu.VMEM((C,), jnp.int32) for _ in range(MI)],
        [pltpu.VMEM((C,), jnp.float32) for _ in range(MV)],
        [pltpu.VMEM((C, D), jnp.float32) for _ in range(MR)],
        [pltpu.VMEM((C, D), jnp.float32) for _ in range(MR)],
        pltpu.VMEM((TAIL,), jnp.int32),
        pltpu.VMEM((TAIL,), jnp.int32),
        pltpu.VMEM((TAIL,), jnp.float32),
        pltpu.VMEM_SHARED((N_NODES, D), jnp.float32),
        [pltpu.SemaphoreType.DMA for _ in range(MI)],
        [pltpu.SemaphoreType.DMA for _ in range(MR)],
        [pltpu.SemaphoreType.DMA for _ in range(MR)],
    ],
    compiler_params=pltpu.CompilerParams(needs_layout_passes=False),
)
def _sc_propagate(emb, src, dst, ev, out, sidx, didx, evv, srows, drows,
                  tsi, tdi, tev, acc, sem_m, sem_g, sem_s):
    cid = lax.axis_index("c")
    sid = lax.axis_index("s")
    wid = sid * NC + cid
    ebase = wid * E_PER_W
    lane = lax.iota(jnp.int32, L)
    zero16 = jnp.zeros((L,), jnp.float32)

    for i in range(C):
        for j in range(D // L):
            srows[0][i, pl.ds(j * L, L)] = zero16
    row0 = sid * 624
    for i in range(624 // C):
        pltpu.sync_copy(srows[0], acc.at[pl.ds(row0 + i * C, C)])
    pltpu.sync_copy(srows[0].at[pl.ds(0, 624 - (624 // C) * C)],
                    acc.at[pl.ds(row0 + (624 // C) * C, 624 - (624 // C) * C)])

    @pl.when(sid == NS - 1)
    def _zero_tail():
        pltpu.sync_copy(srows[0].at[pl.ds(0, 16)], acc.at[pl.ds(9984, 16)])

    plsc.subcore_barrier()

    def issue_ids(c, mi, mv):
        off = ebase + c * C
        pltpu.async_copy(src.at[pl.ds(off, C)], sidx[mi], sem_m[mi])
        pltpu.async_copy(dst.at[pl.ds(off, C)], didx[mi], sem_m[mi])
        pltpu.async_copy(ev.at[pl.ds(off, C)], evv[mv], sem_m[mi])

    def wait_ids(c, mi, mv):
        off = ebase + c * C
        pltpu.make_async_copy(src.at[pl.ds(off, C)], sidx[mi], sem_m[mi]).wait()
        pltpu.make_async_copy(dst.at[pl.ds(off, C)], didx[mi], sem_m[mi]).wait()
        pltpu.make_async_copy(ev.at[pl.ds(off, C)], evv[mv], sem_m[mi]).wait()

    def issue_gathers(mi, mr):
        for q in range(4):
            qs = pl.ds(q * L, L)
            pltpu.async_copy(emb.at[sidx[mi].at[qs]], srows[mr].at[qs],
                             sem_g[mr])
            pltpu.async_copy(emb.at[didx[mi].at[qs]], drows[mr].at[qs],
                             sem_g[mr])

    def wait_gathers(mi, mr):
        for q in range(4):
            qs = pl.ds(q * L, L)
            pltpu.make_async_copy(emb.at[sidx[mi].at[qs]], srows[mr].at[qs],
                                  sem_g[mr]).wait()
            pltpu.make_async_copy(emb.at[didx[mi].at[qs]], drows[mr].at[qs],
                                  sem_g[mr]).wait()

    def issue_scatter(mi, mr):
        pltpu.async_copy(srows[mr], acc.at[sidx[mi]], sem_s[mr], add=True)

    def wait_scatter(mi, mr):
        pltpu.make_async_copy(srows[mr], acc.at[sidx[mi]], sem_s[mr]).wait()

    def softplus_mat(dot, evvec):
        t = 4.0 - 4.0 * dot
        u = jnp.exp(-jnp.abs(t))
        w = u * (1.0 + u * (-0.5 + u * (1.0 / 3.0 + u * -0.25)))
        for _ in range(3):
            w = w - 1.0 + (1.0 + u) * jnp.exp(-w)
        sp = jnp.maximum(t, 0.0) + w
        return (0.5 * jnp.exp(t) * sp) * evvec

    def compute(mr, mv, groups):
        sr, dr, evr = srows[mr], drows[mr], evv[mv]

        def group_body(g, _):
            evec = lane + g * L

            def dot_body(kk, accs):
                a0, a1 = accs
                for j in range(UNROLL):
                    ks = jnp.full((L,), kk * UNROLL + j, jnp.int32)
                    sv = plsc.load_gather(sr, [evec, ks])
                    dv = plsc.load_gather(dr, [evec, ks])
                    if j % 2 == 0:
                        a0 = a0 + sv * dv
                    else:
                        a1 = a1 + sv * dv
                return a0, a1

            with jax.named_scope("dotl"):
                a0, a1 = lax.fori_loop(0, D // UNROLL, dot_body,
                                       (zero16, zero16))
            with jax.named_scope("spl"):
                dot = (a0 + a1) * (1.0 / D)
                mat = softplus_mat(dot, evr[pl.ds(g * L, L)])

            def msg_body(kk, _):
                for j in range(UNROLL):
                    ks = jnp.full((L,), kk * UNROLL + j, jnp.int32)
                    dv = plsc.load_gather(dr, [evec, ks])
                    plsc.store_scatter(sr, [evec, ks], dv * mat)
                return 0

            with jax.named_scope("msgl"):
                lax.fori_loop(0, D // UNROLL, msg_body, 0)
            return 0

        lax.fori_loop(0, groups, group_body, 0)

    issue_ids(0, 0, 0)
    issue_ids(1, 1, 1)
    wait_ids(0, 0, 0)
    issue_gathers(0, 0)

    def body(i, carry):
        c0 = UC * i
        for j in range(UC):
            c = c0 + j
            mr, mi, mv = j % MR, j % MI, j % MV
            mr1, mi1, mv1 = (j + 1) % MR, (j + 1) % MI, (j + 1) % MV
            mi2, mv2 = (j + 2) % MI, (j + 2) % MV

            if j >= 2:
                wait_scatter((j - 2) % MI, (j - 2) % MR)
            else:
                @pl.when(i > 0)
                def _ws():
                    wait_scatter((j - 2) % MI, (j - 2) % MR)

            with jax.named_scope("stage"):
                @pl.when(c + 1 < NFULL)
                def _g1():
                    wait_ids(c + 1, mi1, mv1)
                    issue_gathers(mi1, mr1)

                @pl.when(c + 2 < NFULL)
                def _s2():
                    issue_ids(c + 2, mi2, mv2)

            with jax.named_scope("wgath"):
                wait_gathers(mi, mr)
            with jax.named_scope("comp"):
                compute(mr, mv, GROUPS)
            with jax.named_scope("scat"):
                issue_scatter(mi, mr)
        return carry

    lax.fori_loop(0, NFULL // UC, body, 0)
    wait_scatter((NFULL - 2) % MI, (NFULL - 2) % MR)
    wait_scatter((NFULL - 1) % MI, (NFULL - 1) % MR)

    toff = ebase + NFULL * C
    pltpu.sync_copy(src.at[pl.ds(toff, TAIL)], tsi)
    pltpu.sync_copy(dst.at[pl.ds(toff, TAIL)], tdi)
    pltpu.sync_copy(ev.at[pl.ds(toff, TAIL)], tev)
    cp_s = pltpu.async_copy(emb.at[tsi], srows[0].at[pl.ds(0, TAIL)], sem_g[0])
    cp_d = pltpu.async_copy(emb.at[tdi], drows[0].at[pl.ds(0, TAIL)], sem_g[0])
    cp_s.wait()
    cp_d.wait()
    evec = lane

    def tail_dot(kk, accs):
        a0, a1 = accs
        for j in range(UNROLL):
            ks = jnp.full((L,), kk * UNROLL + j, jnp.int32)
            sv = plsc.load_gather(srows[0], [evec, ks])
            dv = plsc.load_gather(drows[0], [evec, ks])
            if j % 2 == 0:
                a0 = a0 + sv * dv
            else:
                a1 = a1 + sv * dv
        return a0, a1

    a0, a1 = lax.fori_loop(0, D // UNROLL, tail_dot, (zero16, zero16))
    mat = softplus_mat((a0 + a1) * (1.0 / D), tev[...])

    def tail_msg(kk, _):
        for j in range(UNROLL):
            ks = jnp.full((L,), kk * UNROLL + j, jnp.int32)
            dv = plsc.load_gather(drows[0], [evec, ks])
            plsc.store_scatter(srows[0], [evec, ks], dv * mat)
        return 0

    lax.fori_loop(0, D // UNROLL, tail_msg, 0)
    pltpu.sync_copy(srows[0].at[pl.ds(0, TAIL)], acc.at[tsi], add=True)

    plsc.subcore_barrier()

    pltpu.sync_copy(acc.at[pl.ds(row0, 624)], out.at[cid, pl.ds(row0, 624)])

    @pl.when(sid == NS - 1)
    def _write_tail():
        pltpu.sync_copy(acc.at[pl.ds(9984, 16)], out.at[cid, pl.ds(9984, 16)])


def _combine_body(p_ref, o_ref):
    o_ref[...] = p_ref[0] + p_ref[1]


def _combine(parts):
    return pl.pallas_call(
        _combine_body,
        out_shape=jax.ShapeDtypeStruct((N_NODES, D), jnp.float32),
        grid=(10,),
        in_specs=[pl.BlockSpec((NC, N_NODES // 10, D), lambda i: (0, i, 0))],
        out_specs=pl.BlockSpec((N_NODES // 10, D), lambda i: (i, 0)),
    )(parts)


def kernel(emb, edge_index, edge_values):
    src = edge_index[0]
    dst = edge_index[1]
    parts = _sc_propagate(emb, src, dst, edge_values)
    return _combine(parts)


# speedup vs baseline: 12.8553x; 12.8481x over previous
import functools

import jax
import jax.numpy as jnp
from jax import lax
from jax.experimental import pallas as pl
from jax.experimental.pallas import tpu as pltpu
from jax.experimental.pallas import tpu_sc as plsc

N_NODES = 10000
N_EDGES = 320000
D = 128
L = 16
NC = 2
NS = 16
NW = NC * NS
E_PER_W = N_EDGES // NW
C = 64
NFULL = E_PER_W // C
TAIL = E_PER_W - NFULL * C
GROUPS = C // L
UNROLL = 8
MR = 3
MI = 4
MV = 3
UC = 12

_GDN = lax.GatherDimensionNumbers(
    offset_dims=(), collapsed_slice_dims=(0,), start_index_map=(0,)
)


def _shuffle(v, idx):
    return lax.gather(v, idx[:, None], _GDN, slice_sizes=(1,),
                      mode=lax.GatherScatterMode.PROMISE_IN_BOUNDS)


_mesh = plsc.VectorSubcoreMesh(
    core_axis_name="c", subcore_axis_name="s", num_cores=NC, num_subcores=NS
)


@functools.partial(
    pl.kernel,
    out_type=jax.ShapeDtypeStruct((NC, N_NODES, D), jnp.float32),
    mesh=_mesh,
    scratch_types=[
        [pltpu.VMEM((C,), jnp.int32) for _ in range(MI)],
        [pltpu.VMEM((C,), jnp.int32) for _ in range(MI)],
        [pltpu.VMEM((C,), jnp.float32) for _ in range(MV)],
        [pltpu.VMEM((C, D), jnp.float32) for _ in range(MR)],
        [pltpu.VMEM((C, D), jnp.float32) for _ in range(MR)],
        pltpu.VMEM((TAIL,), jnp.int32),
        pltpu.VMEM((TAIL,), jnp.int32),
        pltpu.VMEM((TAIL,), jnp.float32),
        pltpu.VMEM_SHARED((N_NODES, D), jnp.float32),
        [pltpu.SemaphoreType.DMA for _ in range(MI)],
        [pltpu.SemaphoreType.DMA for _ in range(MR)],
        [pltpu.SemaphoreType.DMA for _ in range(MR)],
    ],
    compiler_params=pltpu.CompilerParams(needs_layout_passes=False),
)
def _sc_propagate(emb, src, dst, ev, out, sidx, didx, evv, srows, drows,
                  tsi, tdi, tev, acc, sem_m, sem_g, sem_s):
    cid = lax.axis_index("c")
    sid = lax.axis_index("s")
    wid = sid * NC + cid
    ebase = wid * E_PER_W
    lane = lax.iota(jnp.int32, L)
    zero16 = jnp.zeros((L,), jnp.float32)

    for i in range(C):
        for j in range(D // L):
            srows[0][i, pl.ds(j * L, L)] = zero16
    row0 = sid * 624
    for i in range(624 // C):
        pltpu.sync_copy(srows[0], acc.at[pl.ds(row0 + i * C, C)])
    pltpu.sync_copy(srows[0].at[pl.ds(0, 624 - (624 // C) * C)],
                    acc.at[pl.ds(row0 + (624 // C) * C, 624 - (624 // C) * C)])

    @pl.when(sid == NS - 1)
    def _zero_tail():
        pltpu.sync_copy(srows[0].at[pl.ds(0, 16)], acc.at[pl.ds(9984, 16)])

    plsc.subcore_barrier()

    def issue_ids(c, mi, mv):
        off = ebase + c * C
        pltpu.async_copy(src.at[pl.ds(off, C)], sidx[mi], sem_m[mi])
        pltpu.async_copy(dst.at[pl.ds(off, C)], didx[mi], sem_m[mi])
        pltpu.async_copy(ev.at[pl.ds(off, C)], evv[mv], sem_m[mi])

    def wait_ids(c, mi, mv):
        off = ebase + c * C
        pltpu.make_async_copy(src.at[pl.ds(off, C)], sidx[mi], sem_m[mi]).wait()
        pltpu.make_async_copy(dst.at[pl.ds(off, C)], didx[mi], sem_m[mi]).wait()
        pltpu.make_async_copy(ev.at[pl.ds(off, C)], evv[mv], sem_m[mi]).wait()

    def issue_gathers(mi, mr):
        for q in range(4):
            qs = pl.ds(q * L, L)
            pltpu.async_copy(emb.at[sidx[mi].at[qs]], srows[mr].at[qs],
                             sem_g[mr])
            pltpu.async_copy(emb.at[didx[mi].at[qs]], drows[mr].at[qs],
                             sem_g[mr])

    def wait_gathers(mi, mr):
        for q in range(4):
            qs = pl.ds(q * L, L)
            pltpu.make_async_copy(emb.at[sidx[mi].at[qs]], srows[mr].at[qs],
                                  sem_g[mr]).wait()
            pltpu.make_async_copy(emb.at[didx[mi].at[qs]], drows[mr].at[qs],
                                  sem_g[mr]).wait()

    def issue_scatter(mi, mr):
        pltpu.async_copy(srows[mr], acc.at[sidx[mi]], sem_s[mr], add=True)

    def wait_scatter(mi, mr):
        pltpu.make_async_copy(srows[mr], acc.at[sidx[mi]], sem_s[mr]).wait()

    def softplus_mat(dot, evvec):
        t = 4.0 - 4.0 * dot
        u = jnp.exp(-jnp.abs(t))
        w = u * (1.0 + u * (-0.5 + u * (1.0 / 3.0 + u * -0.25)))
        for _ in range(3):
            w = w - 1.0 + (1.0 + u) * jnp.exp(-w)
        sp = jnp.maximum(t, 0.0) + w
        return (0.5 * jnp.exp(t) * sp) * evvec

    def compute_rows(sr, dr, evr, groups):
        def group_body(g, _):
            base = g * L

            with jax.named_scope("dotl"):
                @plsc.parallel_loop(0, L, unroll=2, carry=zero16)
                def dots(e, dacc):
                    row = base + e
                    a0 = sr[row, pl.ds(0, L)] * dr[row, pl.ds(0, L)]
                    a1 = sr[row, pl.ds(L, L)] * dr[row, pl.ds(L, L)]
                    for j in range(2, D // L):
                        sv = sr[row, pl.ds(j * L, L)]
                        dv = dr[row, pl.ds(j * L, L)]
                        if j % 2 == 0:
                            a0 = a0 + sv * dv
                        else:
                            a1 = a1 + sv * dv
                    p = a0 + a1
                    for sh in (1, 2, 4, 8):
                        p = p + _shuffle(p, lane ^ sh)
                    return jnp.where(lane == e, p, dacc)

            with jax.named_scope("spl"):
                mat = softplus_mat(dots * (1.0 / D), evr[pl.ds(base, L)])

            with jax.named_scope("msgl"):
                @plsc.parallel_loop(0, L, unroll=2)
                def _msg(e):
                    row = base + e
                    m = _shuffle(mat, jnp.full((L,), e, jnp.int32))
                    for j in range(D // L):
                        sr[row, pl.ds(j * L, L)] = (
                            dr[row, pl.ds(j * L, L)] * m)
            return 0

        lax.fori_loop(0, groups, group_body, 0)

    def compute(mr, mv, groups):
        compute_rows(srows[mr], drows[mr], evv[mv], groups)

    issue_ids(0, 0, 0)
    issue_ids(1, 1, 1)
    wait_ids(0, 0, 0)
    issue_gathers(0, 0)

    def body(i, carry):
        c0 = UC * i
        for j in range(UC):
            c = c0 + j
            mr, mi, mv = j % MR, j % MI, j % MV
            mr1, mi1, mv1 = (j + 1) % MR, (j + 1) % MI, (j + 1) % MV
            mi2, mv2 = (j + 2) % MI, (j + 2) % MV

            if j >= 2:
                wait_scatter((j - 2) % MI, (j - 2) % MR)
            else:
                @pl.when(i > 0)
                def _ws():
                    wait_scatter((j - 2) % MI, (j - 2) % MR)

            with jax.named_scope("stage"):
                @pl.when(c + 1 < NFULL)
                def _g1():
                    wait_ids(c + 1, mi1, mv1)
                    issue_gathers(mi1, mr1)

                @pl.when(c + 2 < NFULL)
                def _s2():
                    issue_ids(c + 2, mi2, mv2)

            with jax.named_scope("wgath"):
                wait_gathers(mi, mr)
            with jax.named_scope("comp"):
                compute(mr, mv, GROUPS)
            with jax.named_scope("scat"):
                issue_scatter(mi, mr)
        return carry

    lax.fori_loop(0, NFULL // UC, body, 0)
    wait_scatter((NFULL - 2) % MI, (NFULL - 2) % MR)
    wait_scatter((NFULL - 1) % MI, (NFULL - 1) % MR)

    toff = ebase + NFULL * C
    pltpu.sync_copy(src.at[pl.ds(toff, TAIL)], tsi)
    pltpu.sync_copy(dst.at[pl.ds(toff, TAIL)], tdi)
    pltpu.sync_copy(ev.at[pl.ds(toff, TAIL)], tev)
    cp_s = pltpu.async_copy(emb.at[tsi], srows[0].at[pl.ds(0, TAIL)], sem_g[0])
    cp_d = pltpu.async_copy(emb.at[tdi], drows[0].at[pl.ds(0, TAIL)], sem_g[0])
    cp_s.wait()
    cp_d.wait()
    compute_rows(srows[0], drows[0], tev, 1)
    pltpu.sync_copy(srows[0].at[pl.ds(0, TAIL)], acc.at[tsi], add=True)

    plsc.subcore_barrier()

    pltpu.sync_copy(acc.at[pl.ds(row0, 624)], out.at[cid, pl.ds(row0, 624)])

    @pl.when(sid == NS - 1)
    def _write_tail():
        pltpu.sync_copy(acc.at[pl.ds(9984, 16)], out.at[cid, pl.ds(9984, 16)])


def _combine_body(p_ref, o_ref):
    o_ref[...] = p_ref[0] + p_ref[1]


def _combine(parts):
    return pl.pallas_call(
        _combine_body,
        out_shape=jax.ShapeDtypeStruct((N_NODES, D), jnp.float32),
        grid=(10,),
        in_specs=[pl.BlockSpec((NC, N_NODES // 10, D), lambda i: (0, i, 0))],
        out_specs=pl.BlockSpec((N_NODES // 10, D), lambda i: (i, 0)),
    )(parts)


def kernel(emb, edge_index, edge_values):
    src = edge_index[0]
    dst = edge_index[1]
    parts = _sc_propagate(emb, src, dst, edge_values)
    return _combine(parts)
